# baseline (speedup 1.0000x reference)
import functools

import jax
import jax.numpy as jnp
from jax import lax
from jax.experimental import pallas as pl
from jax.experimental.pallas import tpu as pltpu
from jax.experimental.pallas import tpu_sc as plsc

N0, N1, N2, N3 = 100000, 20000, 5000, 1024
E1, E2, E3 = 320000, 80000, 16384
D = 128
NC, NS = 2, 16
CHUNK = 128


def _ceil_to(v, m):
    return (v + m - 1) // m * m


def _make_seg_sum(ndst, e_pad):
    half = ndst // NC
    half_out = _ceil_to(half, 128)
    garbage = half_out
    acc_rows = _ceil_to(half_out + 1, 128)
    cnt_len = _ceil_to(half_out + 1, 256)
    cpt = e_pad // (NS * CHUNK)
    assert e_pad == cpt * NS * CHUNK and cpt >= 2
    stripe_rows = acc_rows // NS
    assert stripe_rows % 8 == 0
    zn16, zrm = stripe_rows // 16, stripe_rows % 16
    cnt_stripe = cnt_len // NS
    assert cnt_stripe % 16 == 0
    rpt = half_out // NS
    assert rpt % 8 == 0

    mesh = plsc.VectorSubcoreMesh(core_axis_name="c", subcore_axis_name="s",
                                  num_cores=NC, num_subcores=NS)

    @functools.partial(
        pl.kernel,
        out_type=(jax.ShapeDtypeStruct((NC * half_out, D), jnp.float32),
                  jax.ShapeDtypeStruct((NC * half_out,), jnp.float32)),
        mesh=mesh,
        scratch_types=[
            pltpu.VMEM((2, CHUNK), jnp.int32),
            pltpu.VMEM((2, CHUNK), jnp.int32),
            pltpu.VMEM((2, CHUNK, D), jnp.float32),
            pltpu.VMEM((CHUNK,), jnp.float32),
            pltpu.VMEM((16, D), jnp.float32),
            pltpu.VMEM((cnt_stripe,), jnp.float32),
            pltpu.VMEM((rpt,), jnp.float32),
            pltpu.SemaphoreType.DMA((2,)),
            pltpu.SemaphoreType.DMA((2,)),
            pltpu.SemaphoreType.DMA,
            pltpu.VMEM_SHARED((acc_rows, D), jnp.float32),
            pltpu.VMEM_SHARED((cnt_len,), jnp.float32),
        ],
    )
    def seg(src_hbm, dst_hbm, x_hbm, out_hbm, cnt_hbm,
            idx_v, dloc_v, rows_v, ones_v, zrow_v, zcnt_v, cbuf_v,
            sem_g, sem_s, sem_z, acc_sh, cnt_sh):
        c = lax.axis_index("c")
        s = lax.axis_index("s")
        zf16 = jnp.zeros((16,), jnp.float32)
        o16 = jnp.ones((16,), jnp.float32)
        for r in range(16):
            for j in range(D // 16):
                zrow_v[r, pl.ds(j * 16, 16)] = zf16
        for j in range(CHUNK // 16):
            ones_v[pl.ds(j * 16, 16)] = o16
        for j in range(cnt_stripe // 16):
            zcnt_v[pl.ds(j * 16, 16)] = zf16

        zbase = s * stripe_rows
        for q in range(zn16):
            pltpu.async_copy(zrow_v, acc_sh.at[pl.ds(zbase + q * 16, 16)],
                             sem_z)
        if zrm:
            pltpu.async_copy(zrow_v.at[pl.ds(0, zrm)],
                             acc_sh.at[pl.ds(zbase + zn16 * 16, zrm)], sem_z)
        pltpu.async_copy(zcnt_v, cnt_sh.at[pl.ds(s * cnt_stripe, cnt_stripe)],
                         sem_z)
        for q in range(zn16):
            pltpu.make_async_copy(
                zrow_v, acc_sh.at[pl.ds(zbase + q * 16, 16)], sem_z).wait()
        if zrm:
            pltpu.make_async_copy(
                zrow_v.at[pl.ds(0, zrm)],
                acc_sh.at[pl.ds(zbase + zn16 * 16, zrm)], sem_z).wait()
        pltpu.make_async_copy(
            zcnt_v, cnt_sh.at[pl.ds(s * cnt_stripe, cnt_stripe)],
            sem_z).wait()
        plsc.subcore_barrier()

        lo = c * half
        hi = lo + half

        def stage(k, b):
            base = (s * cpt + k) * CHUNK
            pltpu.sync_copy(src_hbm.at[pl.ds(base, CHUNK)], idx_v.at[b])
            pltpu.sync_copy(dst_hbm.at[pl.ds(base, CHUNK)], dloc_v.at[b])
            for j in range(CHUNK // 16):
                dd = dloc_v[b, pl.ds(j * 16, 16)]
                oob = (dd < lo) | (dd >= hi)
                dloc_v[b, pl.ds(j * 16, 16)] = \
                    jnp.where(oob, garbage, dd - lo)

        def fire_gather(b):
            pltpu.async_copy(x_hbm.at[idx_v.at[b]], rows_v.at[b],
                             sem_g.at[b])

        def wait_gather(b):
            pltpu.make_async_copy(x_hbm.at[idx_v.at[b]], rows_v.at[b],
                                  sem_g.at[b]).wait()

        def fire_scatter(b):
            pltpu.async_copy(rows_v.at[b], acc_sh.at[dloc_v.at[b]],
                             sem_s.at[b], add=True)
            pltpu.async_copy(ones_v, cnt_sh.at[dloc_v.at[b]],
                             sem_s.at[b], add=True)

        def drain_scatter(b):
            pltpu.make_async_copy(rows_v.at[b], acc_sh.at[dloc_v.at[b]],
                                  sem_s.at[b]).wait()
            pltpu.make_async_copy(ones_v, cnt_sh.at[dloc_v.at[b]],
                                  sem_s.at[b]).wait()

        stage(jnp.int32(0), jnp.int32(0))
        fire_gather(jnp.int32(0))

        def cbody(i, carry):
            b = jnp.bitwise_and(i, 1)
            nb = 1 - b

            @pl.when(i + 1 < cpt)
            def _():
                @pl.when(i >= 1)
                def _():
                    drain_scatter(nb)
                stage(i + 1, nb)
                fire_gather(nb)

            wait_gather(b)
            fire_scatter(b)
            return carry

        lax.fori_loop(0, cpt, cbody, 0)
        drain_scatter(jnp.int32((cpt - 2) % 2))
        drain_scatter(jnp.int32((cpt - 1) % 2))
        plsc.subcore_barrier()

        pltpu.sync_copy(acc_sh.at[pl.ds(s * rpt, rpt)],
                        out_hbm.at[pl.ds(c * half_out + s * rpt, rpt)])
        pltpu.sync_copy(cnt_sh.at[pl.ds(s * rpt, rpt)], cbuf_v)
        pltpu.sync_copy(cbuf_v,
                        cnt_hbm.at[pl.ds(c * half_out + s * rpt, rpt)])

    return seg, half, half_out


_EPAD = {N1: _ceil_to(E1, NS * CHUNK),
         N2: _ceil_to(E2, NS * CHUNK),
         N3: _ceil_to(E3, NS * CHUNK)}
_SEGS = {n: _make_seg_sum(n, _EPAD[n]) for n in (N1, N2, N3)}


def _tc_layer(summed, cnt2, x_tgt, wlT, bl2, wrT, a2, n):
    BR = 512
    grid = (n + BR - 1) // BR

    def body(s_ref, c_ref, xt_ref, wl_ref, bl_ref, wr_ref, a_ref, o_ref):
        ct = c_ref[...]
        mean = s_ref[...] / jnp.maximum(ct, 1.0)
        y = jnp.dot(mean, wl_ref[...], preferred_element_type=jnp.float32,
                    precision=lax.Precision.HIGHEST)
        y = y + jnp.dot(xt_ref[...], wr_ref[...],
                        preferred_element_type=jnp.float32,
                        precision=lax.Precision.HIGHEST)
        y = y + bl_ref[...]
        o_ref[...] = jnp.where(y > 0.0, y, a_ref[...] * y)

    return pl.pallas_call(
        body,
        grid=(grid,),
        in_specs=[
            pl.BlockSpec((BR, D), lambda i: (i, 0)),
            pl.BlockSpec((BR, 1), lambda i: (i, 0)),
            pl.BlockSpec((BR, D), lambda i: (i, 0)),
            pl.BlockSpec((D, D), lambda i: (0, 0)),
            pl.BlockSpec((1, D), lambda i: (0, 0)),
            pl.BlockSpec((D, D), lambda i: (0, 0)),
            pl.BlockSpec((1, D), lambda i: (0, 0)),
        ],
        out_specs=pl.BlockSpec((BR, D), lambda i: (i, 0)),
        out_shape=jax.ShapeDtypeStruct((n, D), jnp.float32),
    )(summed, cnt2, x_tgt, wlT, bl2, wrT, a2)


def _layer(x_src, x_tgt, edge_index, ndst, Wl, bl, Wr, a):
    seg, half, half_out = _SEGS[ndst]
    e_pad = _EPAD[ndst]
    src = edge_index[0]
    dst = edge_index[1]
    padn = e_pad - src.shape[0]
    if padn:
        src = jnp.pad(src, (0, padn))
        dst = jnp.pad(dst, (0, padn), constant_values=-1)
    flat, cntf = seg(src, dst, x_src)
    if half == half_out:
        summed, cnt = flat, cntf
    else:
        summed = jnp.concatenate(
            [flat[:half], flat[half_out:half_out + half]], axis=0)
        cnt = jnp.concatenate([cntf[:half], cntf[half_out:half_out + half]])
    return _tc_layer(summed, cnt[:, None], x_tgt, Wl.T, bl[None, :], Wr.T,
                     a[None, :], ndst)


def kernel(x, edge_index1, edge_index2, edge_index3,
           Wl1, bl1, Wr1, a1, Wl2, bl2, Wr2, a2, Wl3, bl3, Wr3, a3):
    h1 = _layer(x, x[:N1], edge_index1, N1, Wl1, bl1, Wr1, a1)
    h2 = _layer(h1, h1[:N2], edge_index2, N2, Wl2, bl2, Wr2, a2)
    h3 = _layer(h2, h2[:N3], edge_index3, N3, Wl3, bl3, Wr3, a3)
    return h3
